# baseline (speedup 1.0000x reference)
import jax
import jax.numpy as jnp
from jax.experimental import pallas as pl
from jax.experimental.pallas import tpu as pltpu

B, N, C, W, H = 8, 16, 64, 32, 32
TS = 64
ROWS = 4
STEPS = (B * N) // ROWS


def _assemble_kernel(x_ref, t_ref, emb_ref, out_ref, tv_ref):
    i = pl.program_id(0)

    @pl.when(i == 0)
    def _():
        for n in range(N):
            row = emb_ref[t_ref[n], :]
            tv_ref[n] = jax.lax.broadcast_in_dim(row, (TS, W, H), (0,))

    for r in range(ROWS):
        out_ref[r, :C] = x_ref[r]
        out_ref[r, C:] = tv_ref[(ROWS * i + r) % N]


def kernel(x, t, emb_table):
    x3 = x.reshape(B * N, C, W, H)
    out = pl.pallas_call(
        _assemble_kernel,
        grid=(STEPS,),
        in_specs=[
            pl.BlockSpec((ROWS, C, W, H), lambda i: (i, 0, 0, 0)),
            pl.BlockSpec(memory_space=pltpu.SMEM),
            pl.BlockSpec(memory_space=pltpu.VMEM),
        ],
        out_specs=pl.BlockSpec((ROWS, C + TS, W, H), lambda i: (i, 0, 0, 0)),
        out_shape=jax.ShapeDtypeStruct((B * N, C + TS, W, H), x.dtype),
        scratch_shapes=[pltpu.VMEM((N, TS, W, H), x.dtype)],
    )(x3, t.astype(jnp.int32), emb_table)
    return out.reshape(B, N, C + TS, W, H)
